# baseline (speedup 1.0000x reference)
import functools

import jax
import jax.numpy as jnp
from jax import lax
from jax.experimental import pallas as pl
from jax.experimental.pallas import tpu as pltpu

D_MODEL = 2048
D_FF = 5632
NUM_EXPERTS = 8
TOP_K = 2

TM = 256
FB = 256
NF = D_FF // FB
P_CAP = TOP_K * 2048 + NUM_EXPERTS * TM
NT = P_CAP // TM


def _h_body(te_ref, valid_ref, x_hbm, wg_ref, bg_ref, wu_ref, bu_ref,
            h_ref, x_vmem, sem_in):
    f = pl.program_id(0)
    m = pl.program_id(1)

    @pl.when(jnp.logical_and(f == 0, m == 0))
    def _():
        pltpu.make_async_copy(x_hbm, x_vmem, sem_in).start()
        pltpu.make_async_copy(x_hbm, x_vmem, sem_in).wait()

    @pl.when(valid_ref[m] != 0)
    def _():
        dn = (((1,), (1,)), ((), ()))
        x = x_vmem[pl.ds(m * TM, TM), :]
        h1 = lax.dot_general(x, wg_ref[0], dn,
                             preferred_element_type=jnp.float32) + bg_ref[0]
        h1 = h1 * jax.nn.sigmoid(h1)
        h2 = lax.dot_general(x, wu_ref[0], dn,
                             preferred_element_type=jnp.float32) + bu_ref[0]
        h_ref[...] = (h1 * h2).astype(jnp.bfloat16)


def _y_body(te_ref, valid_ref, h_ref, wd_ref, bd_ref, out_hbm,
            y_vmem, sem_out):
    f = pl.program_id(0)
    m = pl.program_id(1)
    rows = pl.ds(m * TM, TM)

    @pl.when(valid_ref[m] != 0)
    def _():
        dn = (((1,), (1,)), ((), ()))
        h = h_ref[...].astype(jnp.float32)
        y = lax.dot_general(h, wd_ref[0], dn,
                            preferred_element_type=jnp.float32)

        @pl.when(f == 0)
        def _():
            y_vmem[rows, :] = y + bd_ref[0]

        @pl.when(f != 0)
        def _():
            y_vmem[rows, :] = y_vmem[rows, :] + y

        @pl.when(f == NF - 1)
        def _():
            cp = pltpu.make_async_copy(y_vmem.at[rows, :],
                                       out_hbm.at[rows, :], sem_out)
            cp.start()
            cp.wait()


def _grouped_mlp(tile_expert, tile_valid, xg, Wg, bg, Wu, bu, Wd, bd):
    h_spec = pltpu.PrefetchScalarGridSpec(
        num_scalar_prefetch=2,
        grid=(NF, NT),
        in_specs=[
            pl.BlockSpec(memory_space=pl.ANY),
            pl.BlockSpec((1, FB, D_MODEL), lambda f, m, te, va: (te[m], f, 0)),
            pl.BlockSpec((1, 1, FB), lambda f, m, te, va: (te[m], 0, f)),
            pl.BlockSpec((1, FB, D_MODEL), lambda f, m, te, va: (te[m], f, 0)),
            pl.BlockSpec((1, 1, FB), lambda f, m, te, va: (te[m], 0, f)),
        ],
        out_specs=pl.BlockSpec((TM, FB), lambda f, m, te, va: (m, f)),
        scratch_shapes=[
            pltpu.VMEM((P_CAP, D_MODEL), jnp.float32),
            pltpu.SemaphoreType.DMA,
        ],
    )
    h = pl.pallas_call(
        _h_body,
        grid_spec=h_spec,
        out_shape=jax.ShapeDtypeStruct((P_CAP, D_FF), jnp.bfloat16),
        compiler_params=pltpu.CompilerParams(
            dimension_semantics=("arbitrary", "arbitrary")),
    )(tile_expert, tile_valid, xg, Wg, bg, Wu, bu)

    y_spec = pltpu.PrefetchScalarGridSpec(
        num_scalar_prefetch=2,
        grid=(NF, NT),
        in_specs=[
            pl.BlockSpec((TM, FB), lambda f, m, te, va: (m, f)),
            pl.BlockSpec((1, D_MODEL, FB), lambda f, m, te, va: (te[m], 0, f)),
            pl.BlockSpec((1, 1, D_MODEL), lambda f, m, te, va: (te[m], 0, 0)),
        ],
        out_specs=pl.BlockSpec(memory_space=pl.ANY),
        scratch_shapes=[
            pltpu.VMEM((P_CAP, D_MODEL), jnp.float32),
            pltpu.SemaphoreType.DMA,
        ],
    )
    return pl.pallas_call(
        _y_body,
        grid_spec=y_spec,
        out_shape=jax.ShapeDtypeStruct((P_CAP, D_MODEL), jnp.float32),
        compiler_params=pltpu.CompilerParams(
            dimension_semantics=("arbitrary", "arbitrary")),
    )(tile_expert, tile_valid, h, Wd, bd)


def kernel(x, gate_w, gate_b, Wg, bg, Wu, bu, Wd, bd):
    b, s, d = x.shape
    T = b * s
    xf = x.reshape(T, d)

    logits = xf @ gate_w.T + gate_b
    probs = jax.nn.softmax(logits.astype(jnp.float32), axis=1)
    rw, sel = lax.top_k(probs, TOP_K)
    rw = rw / jnp.sum(rw, axis=-1, keepdims=True)

    flat_e = sel.reshape(-1).astype(jnp.int32)
    A = flat_e.shape[0]
    sort_idx = jnp.argsort(flat_e, stable=True)
    inv = jnp.zeros((A,), jnp.int32).at[sort_idx].set(
        jnp.arange(A, dtype=jnp.int32))
    counts = jnp.bincount(flat_e, length=NUM_EXPERTS).astype(jnp.int32)
    raw_off = jnp.concatenate(
        [jnp.zeros((1,), jnp.int32), jnp.cumsum(counts)[:-1]])
    pad_counts = ((counts + TM - 1) // TM) * TM
    pad_end = jnp.cumsum(pad_counts)
    pad_off = jnp.concatenate([jnp.zeros((1,), jnp.int32), pad_end[:-1]])
    pos = pad_off[flat_e] + (inv - raw_off[flat_e])
    tok = (jnp.arange(A, dtype=jnp.int32) // TOP_K)
    perm_tok = jnp.zeros((P_CAP,), jnp.int32).at[pos].set(tok)
    xg = xf[perm_tok]

    tile_start = jnp.arange(NT, dtype=jnp.int32) * TM
    te = jnp.searchsorted(pad_end, tile_start, side='right').astype(jnp.int32)
    te = jnp.minimum(te, NUM_EXPERTS - 1)
    valid = (tile_start < pad_end[-1]).astype(jnp.int32)

    y = _grouped_mlp(te, valid, xg, Wg, bg.reshape(NUM_EXPERTS, 1, D_FF),
                     Wu, bu.reshape(NUM_EXPERTS, 1, D_FF),
                     Wd, bd.reshape(NUM_EXPERTS, 1, D_MODEL))

    p = pos.reshape(T, TOP_K)
    final = rw[:, 0:1] * y[p[:, 0]] + rw[:, 1:2] * y[p[:, 1]]
    return final.reshape(b, s, d), logits


# speedup vs baseline: 1.2861x; 1.2861x over previous
import functools

import jax
import jax.numpy as jnp
from jax import lax
from jax.experimental import pallas as pl
from jax.experimental.pallas import tpu as pltpu

D_MODEL = 2048
D_FF = 5632
NUM_EXPERTS = 8
TOP_K = 2

TM = 256
FB1 = 512
NF1 = D_FF // FB1
FB2 = 512
NF2 = D_FF // FB2
P_CAP = TOP_K * 2048 + NUM_EXPERTS * TM
NT = P_CAP // TM


def _h_body(te_ref, valid_ref, x_hbm, wg_ref, bg_ref, wu_ref, bu_ref,
            h_ref, x_vmem, sem_in):
    f = pl.program_id(0)
    m = pl.program_id(1)

    @pl.when(jnp.logical_and(f == 0, m == 0))
    def _():
        pltpu.make_async_copy(x_hbm, x_vmem, sem_in).start()
        pltpu.make_async_copy(x_hbm, x_vmem, sem_in).wait()

    @pl.when(valid_ref[m] != 0)
    def _():
        dn = (((1,), (1,)), ((), ()))
        x = x_vmem[pl.ds(m * TM, TM), :].astype(jnp.float32)
        h1 = lax.dot_general(x, wg_ref[0], dn,
                             preferred_element_type=jnp.float32) + bg_ref[0]
        h1 = h1 * jax.nn.sigmoid(h1)
        h2 = lax.dot_general(x, wu_ref[0], dn,
                             preferred_element_type=jnp.float32) + bu_ref[0]
        h_ref[...] = (h1 * h2).astype(jnp.bfloat16)


def _y_body(te_ref, valid_ref, h_ref, wd_ref, bd_ref, out_hbm,
            y_vmem, sem_out):
    f = pl.program_id(0)
    m = pl.program_id(1)
    rows = pl.ds(m * TM, TM)

    @pl.when(valid_ref[m] != 0)
    def _():
        dn = (((1,), (1,)), ((), ()))
        h = h_ref[...].astype(jnp.float32)
        y = lax.dot_general(h, wd_ref[0], dn,
                            preferred_element_type=jnp.float32)

        @pl.when(f == 0)
        def _():
            y_vmem[rows, :] = y + bd_ref[0]

        @pl.when(f != 0)
        def _():
            y_vmem[rows, :] = y_vmem[rows, :] + y

        @pl.when(f == NF2 - 1)
        def _():
            cp = pltpu.make_async_copy(y_vmem.at[rows, :],
                                       out_hbm.at[rows, :], sem_out)
            cp.start()
            cp.wait()


def _grouped_mlp(tile_expert, tile_valid, xg, Wg, bg, Wu, bu, Wd, bd):
    h_spec = pltpu.PrefetchScalarGridSpec(
        num_scalar_prefetch=2,
        grid=(NF1, NT),
        in_specs=[
            pl.BlockSpec(memory_space=pl.ANY),
            pl.BlockSpec((1, FB1, D_MODEL), lambda f, m, te, va: (te[m], f, 0)),
            pl.BlockSpec((1, 1, FB1),
                         lambda f, m, te, va: (te[m] * NF1 + f, 0, 0)),
            pl.BlockSpec((1, FB1, D_MODEL), lambda f, m, te, va: (te[m], f, 0)),
            pl.BlockSpec((1, 1, FB1),
                         lambda f, m, te, va: (te[m] * NF1 + f, 0, 0)),
        ],
        out_specs=pl.BlockSpec((TM, FB1), lambda f, m, te, va: (m, f)),
        scratch_shapes=[
            pltpu.VMEM((P_CAP, D_MODEL), jnp.bfloat16),
            pltpu.SemaphoreType.DMA,
        ],
    )
    h = pl.pallas_call(
        _h_body,
        grid_spec=h_spec,
        out_shape=jax.ShapeDtypeStruct((P_CAP, D_FF), jnp.bfloat16),
        compiler_params=pltpu.CompilerParams(
            dimension_semantics=("arbitrary", "arbitrary")),
    )(tile_expert, tile_valid, xg, Wg, bg, Wu, bu)

    y_spec = pltpu.PrefetchScalarGridSpec(
        num_scalar_prefetch=2,
        grid=(NF2, NT),
        in_specs=[
            pl.BlockSpec((TM, FB2), lambda f, m, te, va: (m, f)),
            pl.BlockSpec((1, D_MODEL, FB2), lambda f, m, te, va: (te[m], 0, f)),
            pl.BlockSpec((1, 1, D_MODEL), lambda f, m, te, va: (te[m], 0, 0)),
        ],
        out_specs=pl.BlockSpec(memory_space=pl.ANY),
        scratch_shapes=[
            pltpu.VMEM((P_CAP, D_MODEL), jnp.float32),
            pltpu.SemaphoreType.DMA,
        ],
    )
    return pl.pallas_call(
        _y_body,
        grid_spec=y_spec,
        out_shape=jax.ShapeDtypeStruct((P_CAP, D_MODEL), jnp.float32),
        compiler_params=pltpu.CompilerParams(
            dimension_semantics=("arbitrary", "arbitrary")),
    )(tile_expert, tile_valid, h, Wd, bd)


def kernel(x, gate_w, gate_b, Wg, bg, Wu, bu, Wd, bd):
    b, s, d = x.shape
    T = b * s
    xf = x.reshape(T, d)

    logits = xf @ gate_w.T + gate_b
    probs = jax.nn.softmax(logits.astype(jnp.float32), axis=1)
    rw, sel = lax.top_k(probs, TOP_K)
    rw = rw / jnp.sum(rw, axis=-1, keepdims=True)

    flat_e = sel.reshape(-1).astype(jnp.int32)
    A = flat_e.shape[0]
    sort_idx = jnp.argsort(flat_e, stable=True)
    inv = jnp.zeros((A,), jnp.int32).at[sort_idx].set(
        jnp.arange(A, dtype=jnp.int32))
    counts = jnp.bincount(flat_e, length=NUM_EXPERTS).astype(jnp.int32)
    raw_off = jnp.concatenate(
        [jnp.zeros((1,), jnp.int32), jnp.cumsum(counts)[:-1]])
    pad_counts = ((counts + TM - 1) // TM) * TM
    pad_end = jnp.cumsum(pad_counts)
    pad_off = jnp.concatenate([jnp.zeros((1,), jnp.int32), pad_end[:-1]])
    pos = pad_off[flat_e] + (inv - raw_off[flat_e])
    tok = (jnp.arange(A, dtype=jnp.int32) // TOP_K)
    perm_tok = jnp.zeros((P_CAP,), jnp.int32).at[pos].set(tok)
    xg = xf[perm_tok].astype(jnp.bfloat16)

    tile_start = jnp.arange(NT, dtype=jnp.int32) * TM
    te = jnp.searchsorted(pad_end, tile_start, side='right').astype(jnp.int32)
    te = jnp.minimum(te, NUM_EXPERTS - 1)
    valid = (tile_start < pad_end[-1]).astype(jnp.int32)

    y = _grouped_mlp(te, valid, xg,
                     Wg, bg.reshape(NUM_EXPERTS * NF1, 1, FB1),
                     Wu, bu.reshape(NUM_EXPERTS * NF1, 1, FB1),
                     Wd, bd.reshape(NUM_EXPERTS, 1, D_MODEL))

    p = pos.reshape(T, TOP_K)
    final = rw[:, 0:1] * y[p[:, 0]] + rw[:, 1:2] * y[p[:, 1]]
    return final.reshape(b, s, d), logits


# speedup vs baseline: 1.3090x; 1.0178x over previous
import functools

import jax
import jax.numpy as jnp
from jax import lax
from jax.experimental import pallas as pl
from jax.experimental.pallas import tpu as pltpu

D_MODEL = 2048
D_FF = 5632
NUM_EXPERTS = 8
TOP_K = 2

TM = 256
FB1 = 512
NF1 = D_FF // FB1
FB2 = 512
NF2 = D_FF // FB2
P_CAP = TOP_K * 2048 + NUM_EXPERTS * TM
NT = P_CAP // TM


def _h_body(te_ref, valid_ref, x_hbm, wg_ref, bg_ref, wu_ref, bu_ref,
            h_ref, x_vmem, sem_in):
    f = pl.program_id(0)
    m = pl.program_id(1)

    @pl.when(jnp.logical_and(f == 0, m == 0))
    def _():
        pltpu.make_async_copy(x_hbm, x_vmem, sem_in).start()
        pltpu.make_async_copy(x_hbm, x_vmem, sem_in).wait()

    @pl.when(valid_ref[m] != 0)
    def _():
        dn = (((1,), (1,)), ((), ()))
        x = x_vmem[pl.ds(m * TM, TM), :].astype(jnp.float32)
        h1 = lax.dot_general(x, wg_ref[0], dn,
                             preferred_element_type=jnp.float32) + bg_ref[0]
        h1 = h1 * jax.nn.sigmoid(h1)
        h2 = lax.dot_general(x, wu_ref[0], dn,
                             preferred_element_type=jnp.float32) + bu_ref[0]
        h_ref[...] = (h1 * h2).astype(jnp.bfloat16)


def _y_body(te_ref, valid_ref, h_ref, wd_ref, bd_ref, out_hbm,
            y_vmem, sem_out):
    f = pl.program_id(0)
    m = pl.program_id(1)
    rows = pl.ds(m * TM, TM)

    @pl.when(valid_ref[m] != 0)
    def _():
        dn = (((1,), (1,)), ((), ()))
        h = h_ref[...].astype(jnp.float32)
        y = lax.dot_general(h, wd_ref[0], dn,
                            preferred_element_type=jnp.float32)

        @pl.when(f == 0)
        def _():
            y_vmem[rows, :] = y + bd_ref[0]

        @pl.when(f != 0)
        def _():
            y_vmem[rows, :] = y_vmem[rows, :] + y

        @pl.when(f == NF2 - 1)
        def _():
            cp = pltpu.make_async_copy(y_vmem.at[rows, :],
                                       out_hbm.at[rows, :], sem_out)
            cp.start()
            cp.wait()


def _grouped_mlp(tile_expert, tile_valid, xg, Wg, bg, Wu, bu, Wd, bd):
    h_spec = pltpu.PrefetchScalarGridSpec(
        num_scalar_prefetch=2,
        grid=(NF1, NT),
        in_specs=[
            pl.BlockSpec(memory_space=pl.ANY),
            pl.BlockSpec((1, FB1, D_MODEL), lambda f, m, te, va: (te[m], f, 0)),
            pl.BlockSpec((1, 1, FB1),
                         lambda f, m, te, va: (te[m] * NF1 + f, 0, 0)),
            pl.BlockSpec((1, FB1, D_MODEL), lambda f, m, te, va: (te[m], f, 0)),
            pl.BlockSpec((1, 1, FB1),
                         lambda f, m, te, va: (te[m] * NF1 + f, 0, 0)),
        ],
        out_specs=pl.BlockSpec((TM, FB1), lambda f, m, te, va: (m, f)),
        scratch_shapes=[
            pltpu.VMEM((P_CAP, D_MODEL), jnp.bfloat16),
            pltpu.SemaphoreType.DMA,
        ],
    )
    h = pl.pallas_call(
        _h_body,
        grid_spec=h_spec,
        out_shape=jax.ShapeDtypeStruct((P_CAP, D_FF), jnp.bfloat16),
        compiler_params=pltpu.CompilerParams(
            dimension_semantics=("arbitrary", "arbitrary")),
    )(tile_expert, tile_valid, xg, Wg, bg, Wu, bu)

    y_spec = pltpu.PrefetchScalarGridSpec(
        num_scalar_prefetch=2,
        grid=(NF2, NT),
        in_specs=[
            pl.BlockSpec((TM, FB2), lambda f, m, te, va: (m, f)),
            pl.BlockSpec((1, D_MODEL, FB2), lambda f, m, te, va: (te[m], 0, f)),
            pl.BlockSpec((1, 1, D_MODEL), lambda f, m, te, va: (te[m], 0, 0)),
        ],
        out_specs=pl.BlockSpec(memory_space=pl.ANY),
        scratch_shapes=[
            pltpu.VMEM((P_CAP, D_MODEL), jnp.float32),
            pltpu.SemaphoreType.DMA,
        ],
    )
    return pl.pallas_call(
        _y_body,
        grid_spec=y_spec,
        out_shape=jax.ShapeDtypeStruct((P_CAP, D_MODEL), jnp.float32),
        compiler_params=pltpu.CompilerParams(
            dimension_semantics=("arbitrary", "arbitrary")),
    )(tile_expert, tile_valid, h, Wd, bd)


def kernel(x, gate_w, gate_b, Wg, bg, Wu, bu, Wd, bd):
    b, s, d = x.shape
    T = b * s
    xf = x.reshape(T, d)

    logits = xf @ gate_w.T + gate_b
    probs = jax.nn.softmax(logits.astype(jnp.float32), axis=1)
    rw, sel = lax.top_k(probs, TOP_K)
    rw = rw / jnp.sum(rw, axis=-1, keepdims=True)

    A = T * TOP_K
    onehot = jnp.sum(
        (sel[:, :, None] == jnp.arange(NUM_EXPERTS)[None, None, :])
        .astype(jnp.float32), axis=1)
    tri = (jnp.arange(T)[:, None] > jnp.arange(T)[None, :]).astype(jnp.float32)
    rank_before = (tri @ onehot).astype(jnp.int32)
    counts = jnp.sum(onehot, axis=0).astype(jnp.int32)
    pad_counts = ((counts + TM - 1) // TM) * TM
    pad_end = jnp.cumsum(pad_counts)
    pad_off = jnp.concatenate([jnp.zeros((1,), jnp.int32), pad_end[:-1]])
    pos = pad_off[sel] + jnp.take_along_axis(rank_before, sel, axis=1)
    pos = pos.reshape(-1)
    tok = (jnp.arange(A, dtype=jnp.int32) // TOP_K)
    perm_tok = jnp.zeros((P_CAP,), jnp.int32).at[pos].set(tok)
    xg = xf[perm_tok].astype(jnp.bfloat16)

    tile_start = jnp.arange(NT, dtype=jnp.int32) * TM
    te = jnp.searchsorted(pad_end, tile_start, side='right').astype(jnp.int32)
    te = jnp.minimum(te, NUM_EXPERTS - 1)
    valid = (tile_start < pad_end[-1]).astype(jnp.int32)

    y = _grouped_mlp(te, valid, xg,
                     Wg, bg.reshape(NUM_EXPERTS * NF1, 1, FB1),
                     Wu, bu.reshape(NUM_EXPERTS * NF1, 1, FB1),
                     Wd, bd.reshape(NUM_EXPERTS, 1, D_MODEL))

    p = pos.reshape(T, TOP_K)
    final = rw[:, 0:1] * y[p[:, 0]] + rw[:, 1:2] * y[p[:, 1]]
    return final.reshape(b, s, d), logits


# speedup vs baseline: 1.7390x; 1.3285x over previous
import functools

import jax
import jax.numpy as jnp
from jax import lax
from jax.experimental import pallas as pl
from jax.experimental.pallas import tpu as pltpu

D_MODEL = 2048
D_FF = 5632
NUM_EXPERTS = 8
TOP_K = 2

TM = 256
FB = 512
NF = D_FF // FB
NT = TOP_K * 2048 // TM + NUM_EXPERTS - 1
P_CAP = NT * TM
NBUF = 3
NBUF2 = 2


def _h_body(te_ref, valid_ref, isnew_ref, rid_ref, runex_ref, meta_ref,
            x_hbm, wg_hbm, bg_ref, wu_hbm, bu_ref,
            h_ref, x_vmem, wg_buf, wu_buf, ctr, sem_in, sems_g, sems_u):
    f = pl.program_id(0)
    m = pl.program_id(1)
    nblk = meta_ref[0]
    nrun = meta_ref[1]

    def fetch(g):
        fe = g // nrun
        er = runex_ref[g % nrun]
        slot = g % NBUF
        pltpu.make_async_copy(wg_hbm.at[er, pl.ds(fe * FB, FB), :],
                              wg_buf.at[slot], sems_g.at[slot]).start()
        pltpu.make_async_copy(wu_hbm.at[er, pl.ds(fe * FB, FB), :],
                              wu_buf.at[slot], sems_u.at[slot]).start()

    @pl.when(jnp.logical_and(f == 0, m == 0))
    def _():
        pltpu.make_async_copy(x_hbm, x_vmem, sem_in).start()
        fetch(jnp.int32(0))
        fetch(jnp.int32(1))
        ctr[0] = jnp.int32(1)
        pltpu.make_async_copy(x_hbm, x_vmem, sem_in).wait()

    blk = f * nrun + rid_ref[m]
    nxt = ctr[0] + 1

    @pl.when(jnp.logical_and(nxt < nblk, nxt - blk <= NBUF - 1))
    def _():
        fetch(nxt)
        ctr[0] = nxt

    e = te_ref[m]
    slot = blk % NBUF

    @pl.when(isnew_ref[m] != 0)
    def _():
        pltpu.make_async_copy(wg_hbm.at[e, pl.ds(f * FB, FB), :],
                              wg_buf.at[slot], sems_g.at[slot]).wait()
        pltpu.make_async_copy(wu_hbm.at[e, pl.ds(f * FB, FB), :],
                              wu_buf.at[slot], sems_u.at[slot]).wait()

    @pl.when(valid_ref[m] != 0)
    def _():
        dn = (((1,), (1,)), ((), ()))
        x = x_vmem[pl.ds(m * TM, TM), :].astype(jnp.float32)
        h1 = lax.dot_general(x, wg_buf[slot], dn,
                             preferred_element_type=jnp.float32) + bg_ref[0]
        h1 = h1 * jax.nn.sigmoid(h1)
        h2 = lax.dot_general(x, wu_buf[slot], dn,
                             preferred_element_type=jnp.float32) + bu_ref[0]
        h_ref[...] = (h1 * h2).astype(jnp.bfloat16)


def _y_body(te_ref, valid_ref, isnew_ref, rid_ref, runex_ref, meta_ref,
            h_ref, wd_hbm, bd_ref, out_hbm,
            y_vmem, wd_buf, ctr, sems_d, sem_out):
    f = pl.program_id(0)
    m = pl.program_id(1)
    nblk = meta_ref[0]
    nrun = meta_ref[1]
    rows = pl.ds(m * TM, TM)

    def fetch(g):
        fe = g // nrun
        er = runex_ref[g % nrun]
        slot = g % NBUF2
        pltpu.make_async_copy(wd_hbm.at[er, :, pl.ds(fe * FB, FB)],
                              wd_buf.at[slot], sems_d.at[slot]).start()

    @pl.when(jnp.logical_and(f == 0, m == 0))
    def _():
        fetch(jnp.int32(0))
        fetch(jnp.int32(1))
        ctr[0] = jnp.int32(1)

    blk = f * nrun + rid_ref[m]
    nxt = ctr[0] + 1

    @pl.when(jnp.logical_and(nxt < nblk, nxt - blk <= NBUF2 - 1))
    def _():
        fetch(nxt)
        ctr[0] = nxt

    e = te_ref[m]
    slot = blk % NBUF2

    @pl.when(isnew_ref[m] != 0)
    def _():
        pltpu.make_async_copy(wd_hbm.at[e, :, pl.ds(f * FB, FB)],
                              wd_buf.at[slot], sems_d.at[slot]).wait()

    @pl.when(valid_ref[m] != 0)
    def _():
        dn = (((1,), (1,)), ((), ()))
        h = h_ref[...].astype(jnp.float32)
        y = lax.dot_general(h, wd_buf[slot], dn,
                            preferred_element_type=jnp.float32)

        @pl.when(f == 0)
        def _():
            y_vmem[rows, :] = y + bd_ref[0]

        @pl.when(f != 0)
        def _():
            y_vmem[rows, :] = y_vmem[rows, :] + y

        @pl.when(f == NF - 1)
        def _():
            cp = pltpu.make_async_copy(y_vmem.at[rows, :],
                                       out_hbm.at[rows, :], sem_out)
            cp.start()
            cp.wait()


def _grouped_mlp(sched, xg, Wg, bg, Wu, bu, Wd, bd):
    te, valid, isnew, rid, runex, meta = sched
    h_spec = pltpu.PrefetchScalarGridSpec(
        num_scalar_prefetch=6,
        grid=(NF, NT),
        in_specs=[
            pl.BlockSpec(memory_space=pl.ANY),
            pl.BlockSpec(memory_space=pl.ANY),
            pl.BlockSpec((1, 1, FB),
                         lambda f, m, te, va, nw, ri, rx, mt:
                         (te[m] * NF + f, 0, 0)),
            pl.BlockSpec(memory_space=pl.ANY),
            pl.BlockSpec((1, 1, FB),
                         lambda f, m, te, va, nw, ri, rx, mt:
                         (te[m] * NF + f, 0, 0)),
        ],
        out_specs=pl.BlockSpec((TM, FB),
                               lambda f, m, te, va, nw, ri, rx, mt: (m, f)),
        scratch_shapes=[
            pltpu.VMEM((P_CAP, D_MODEL), jnp.bfloat16),
            pltpu.VMEM((NBUF, FB, D_MODEL), jnp.float32),
            pltpu.VMEM((NBUF, FB, D_MODEL), jnp.float32),
            pltpu.SMEM((1,), jnp.int32),
            pltpu.SemaphoreType.DMA,
            pltpu.SemaphoreType.DMA((NBUF,)),
            pltpu.SemaphoreType.DMA((NBUF,)),
        ],
    )
    h = pl.pallas_call(
        _h_body,
        grid_spec=h_spec,
        out_shape=jax.ShapeDtypeStruct((P_CAP, D_FF), jnp.bfloat16),
        compiler_params=pltpu.CompilerParams(
            dimension_semantics=("arbitrary", "arbitrary")),
    )(te, valid, isnew, rid, runex, meta, xg, Wg, bg, Wu, bu)

    y_spec = pltpu.PrefetchScalarGridSpec(
        num_scalar_prefetch=6,
        grid=(NF, NT),
        in_specs=[
            pl.BlockSpec((TM, FB),
                         lambda f, m, te, va, nw, ri, rx, mt: (m, f)),
            pl.BlockSpec(memory_space=pl.ANY),
            pl.BlockSpec((1, 1, D_MODEL),
                         lambda f, m, te, va, nw, ri, rx, mt: (te[m], 0, 0)),
        ],
        out_specs=pl.BlockSpec(memory_space=pl.ANY),
        scratch_shapes=[
            pltpu.VMEM((P_CAP, D_MODEL), jnp.float32),
            pltpu.VMEM((NBUF2, D_MODEL, FB), jnp.float32),
            pltpu.SMEM((1,), jnp.int32),
            pltpu.SemaphoreType.DMA((NBUF2,)),
            pltpu.SemaphoreType.DMA,
        ],
    )
    return pl.pallas_call(
        _y_body,
        grid_spec=y_spec,
        out_shape=jax.ShapeDtypeStruct((P_CAP, D_MODEL), jnp.float32),
        compiler_params=pltpu.CompilerParams(
            dimension_semantics=("arbitrary", "arbitrary")),
    )(te, valid, isnew, rid, runex, meta, h, Wd, bd)


def kernel(x, gate_w, gate_b, Wg, bg, Wu, bu, Wd, bd):
    b, s, d = x.shape
    T = b * s
    xf = x.reshape(T, d)

    logits = xf @ gate_w.T + gate_b
    probs = jax.nn.softmax(logits.astype(jnp.float32), axis=1)
    rw, sel = lax.top_k(probs, TOP_K)
    rw = rw / jnp.sum(rw, axis=-1, keepdims=True)

    A = T * TOP_K
    onehot = jnp.sum(
        (sel[:, :, None] == jnp.arange(NUM_EXPERTS)[None, None, :])
        .astype(jnp.float32), axis=1)
    tri = (jnp.arange(T)[:, None] > jnp.arange(T)[None, :]).astype(jnp.float32)
    rank_before = (tri @ onehot).astype(jnp.int32)
    counts = jnp.sum(onehot, axis=0).astype(jnp.int32)
    pad_counts = ((counts + TM - 1) // TM) * TM
    pad_end = jnp.cumsum(pad_counts)
    pad_off = jnp.concatenate([jnp.zeros((1,), jnp.int32), pad_end[:-1]])
    pos = pad_off[sel] + jnp.take_along_axis(rank_before, sel, axis=1)
    pos = pos.reshape(-1)
    tok = (jnp.arange(A, dtype=jnp.int32) // TOP_K)
    perm_tok = jnp.zeros((P_CAP,), jnp.int32).at[pos].set(tok)
    xg = xf[perm_tok].astype(jnp.bfloat16)

    tile_start = jnp.arange(NT, dtype=jnp.int32) * TM
    last_row = pad_end[-1] - 1
    te = jnp.searchsorted(pad_end, jnp.minimum(tile_start, last_row),
                          side='right').astype(jnp.int32)
    valid = (tile_start < pad_end[-1]).astype(jnp.int32)
    isnew = jnp.concatenate([jnp.ones((1,), jnp.int32),
                             (te[1:] != te[:-1]).astype(jnp.int32)])
    rid = (jnp.cumsum(isnew) - 1).astype(jnp.int32)
    nrun = rid[-1] + 1
    runex = jnp.zeros((NUM_EXPERTS,), jnp.int32).at[rid].set(te)
    meta = jnp.stack([NF * nrun, nrun]).astype(jnp.int32)

    sched = (te, valid, isnew, rid, runex, meta)
    y = _grouped_mlp(sched, xg,
                     Wg, bg.reshape(NUM_EXPERTS * NF, 1, FB),
                     Wu, bu.reshape(NUM_EXPERTS * NF, 1, FB),
                     Wd, bd.reshape(NUM_EXPERTS, 1, D_MODEL))

    p = pos.reshape(T, TOP_K)
    final = rw[:, 0:1] * y[p[:, 0]] + rw[:, 1:2] * y[p[:, 1]]
    return final.reshape(b, s, d), logits


# speedup vs baseline: 1.7427x; 1.0021x over previous
import functools

import jax
import jax.numpy as jnp
from jax import lax
from jax.experimental import pallas as pl
from jax.experimental.pallas import tpu as pltpu

D_MODEL = 2048
D_FF = 5632
NUM_EXPERTS = 8
TOP_K = 2

TM = 256
FB = 512
NF = D_FF // FB
NT = TOP_K * 2048 // TM + NUM_EXPERTS - 1
P_CAP = NT * TM
NBUF = 3
NBUF2 = 2


def _h_body(te_ref, valid_ref, isnew_ref, rid_ref, runex_ref, meta_ref,
            x_hbm, wg_hbm, bg_ref, wu_hbm, bu_ref,
            h_ref, x_vmem, wg_buf, wu_buf, ctr, sem_in, sems_g, sems_u):
    f = pl.program_id(0)
    m = pl.program_id(1)
    nblk = meta_ref[0]
    nrun = meta_ref[1]

    def fetch(g):
        fe = g // nrun
        er = runex_ref[g % nrun]
        slot = g % NBUF
        pltpu.make_async_copy(wg_hbm.at[er, pl.ds(fe * FB, FB), :],
                              wg_buf.at[slot], sems_g.at[slot]).start()
        pltpu.make_async_copy(wu_hbm.at[er, pl.ds(fe * FB, FB), :],
                              wu_buf.at[slot], sems_u.at[slot]).start()

    @pl.when(jnp.logical_and(f == 0, m == 0))
    def _():
        pltpu.make_async_copy(x_hbm, x_vmem, sem_in).start()
        fetch(jnp.int32(0))
        fetch(jnp.int32(1))
        ctr[0] = jnp.int32(1)
        pltpu.make_async_copy(x_hbm, x_vmem, sem_in).wait()

    blk = f * nrun + rid_ref[m]
    nxt = ctr[0] + 1

    @pl.when(jnp.logical_and(nxt < nblk, nxt - blk <= NBUF - 1))
    def _():
        fetch(nxt)
        ctr[0] = nxt

    e = te_ref[m]
    slot = blk % NBUF

    @pl.when(isnew_ref[m] != 0)
    def _():
        pltpu.make_async_copy(wg_hbm.at[e, pl.ds(f * FB, FB), :],
                              wg_buf.at[slot], sems_g.at[slot]).wait()
        pltpu.make_async_copy(wu_hbm.at[e, pl.ds(f * FB, FB), :],
                              wu_buf.at[slot], sems_u.at[slot]).wait()

    @pl.when(valid_ref[m] != 0)
    def _():
        dn = (((1,), (1,)), ((), ()))
        x = x_vmem[pl.ds(m * TM, TM), :]
        h1 = lax.dot_general(x, wg_buf[slot].astype(jnp.bfloat16), dn,
                             preferred_element_type=jnp.float32) + bg_ref[0]
        h1 = h1 * jax.nn.sigmoid(h1)
        h2 = lax.dot_general(x, wu_buf[slot].astype(jnp.bfloat16), dn,
                             preferred_element_type=jnp.float32) + bu_ref[0]
        h_ref[...] = (h1 * h2).astype(jnp.bfloat16)


def _y_body(te_ref, valid_ref, isnew_ref, rid_ref, runex_ref, meta_ref,
            h_ref, wd_hbm, bd_ref, out_hbm,
            y_vmem, wd_buf, ctr, sems_d, sem_out):
    f = pl.program_id(0)
    m = pl.program_id(1)
    nblk = meta_ref[0]
    nrun = meta_ref[1]
    rows = pl.ds(m * TM, TM)

    def fetch(g):
        fe = g // nrun
        er = runex_ref[g % nrun]
        slot = g % NBUF2
        pltpu.make_async_copy(wd_hbm.at[er, :, pl.ds(fe * FB, FB)],
                              wd_buf.at[slot], sems_d.at[slot]).start()

    @pl.when(jnp.logical_and(f == 0, m == 0))
    def _():
        fetch(jnp.int32(0))
        fetch(jnp.int32(1))
        ctr[0] = jnp.int32(1)

    blk = f * nrun + rid_ref[m]
    nxt = ctr[0] + 1

    @pl.when(jnp.logical_and(nxt < nblk, nxt - blk <= NBUF2 - 1))
    def _():
        fetch(nxt)
        ctr[0] = nxt

    e = te_ref[m]
    slot = blk % NBUF2

    @pl.when(isnew_ref[m] != 0)
    def _():
        pltpu.make_async_copy(wd_hbm.at[e, :, pl.ds(f * FB, FB)],
                              wd_buf.at[slot], sems_d.at[slot]).wait()

    @pl.when(valid_ref[m] != 0)
    def _():
        dn = (((1,), (1,)), ((), ()))
        y = lax.dot_general(h_ref[...], wd_buf[slot].astype(jnp.bfloat16),
                            dn, preferred_element_type=jnp.float32)

        @pl.when(f == 0)
        def _():
            y_vmem[rows, :] = y + bd_ref[0]

        @pl.when(f != 0)
        def _():
            y_vmem[rows, :] = y_vmem[rows, :] + y

        @pl.when(f == NF - 1)
        def _():
            cp = pltpu.make_async_copy(y_vmem.at[rows, :],
                                       out_hbm.at[rows, :], sem_out)
            cp.start()
            cp.wait()


def _grouped_mlp(sched, xg, Wg, bg, Wu, bu, Wd, bd):
    te, valid, isnew, rid, runex, meta = sched
    h_spec = pltpu.PrefetchScalarGridSpec(
        num_scalar_prefetch=6,
        grid=(NF, NT),
        in_specs=[
            pl.BlockSpec(memory_space=pl.ANY),
            pl.BlockSpec(memory_space=pl.ANY),
            pl.BlockSpec((1, 1, FB),
                         lambda f, m, te, va, nw, ri, rx, mt:
                         (te[m] * NF + f, 0, 0)),
            pl.BlockSpec(memory_space=pl.ANY),
            pl.BlockSpec((1, 1, FB),
                         lambda f, m, te, va, nw, ri, rx, mt:
                         (te[m] * NF + f, 0, 0)),
        ],
        out_specs=pl.BlockSpec((TM, FB),
                               lambda f, m, te, va, nw, ri, rx, mt: (m, f)),
        scratch_shapes=[
            pltpu.VMEM((P_CAP, D_MODEL), jnp.bfloat16),
            pltpu.VMEM((NBUF, FB, D_MODEL), jnp.float32),
            pltpu.VMEM((NBUF, FB, D_MODEL), jnp.float32),
            pltpu.SMEM((1,), jnp.int32),
            pltpu.SemaphoreType.DMA,
            pltpu.SemaphoreType.DMA((NBUF,)),
            pltpu.SemaphoreType.DMA((NBUF,)),
        ],
    )
    h = pl.pallas_call(
        _h_body,
        grid_spec=h_spec,
        out_shape=jax.ShapeDtypeStruct((P_CAP, D_FF), jnp.bfloat16),
        compiler_params=pltpu.CompilerParams(
            dimension_semantics=("arbitrary", "arbitrary")),
    )(te, valid, isnew, rid, runex, meta, xg, Wg, bg, Wu, bu)

    y_spec = pltpu.PrefetchScalarGridSpec(
        num_scalar_prefetch=6,
        grid=(NF, NT),
        in_specs=[
            pl.BlockSpec((TM, FB),
                         lambda f, m, te, va, nw, ri, rx, mt: (m, f)),
            pl.BlockSpec(memory_space=pl.ANY),
            pl.BlockSpec((1, 1, D_MODEL),
                         lambda f, m, te, va, nw, ri, rx, mt: (te[m], 0, 0)),
        ],
        out_specs=pl.BlockSpec(memory_space=pl.ANY),
        scratch_shapes=[
            pltpu.VMEM((P_CAP, D_MODEL), jnp.float32),
            pltpu.VMEM((NBUF2, D_MODEL, FB), jnp.float32),
            pltpu.SMEM((1,), jnp.int32),
            pltpu.SemaphoreType.DMA((NBUF2,)),
            pltpu.SemaphoreType.DMA,
        ],
    )
    return pl.pallas_call(
        _y_body,
        grid_spec=y_spec,
        out_shape=jax.ShapeDtypeStruct((P_CAP, D_MODEL), jnp.float32),
        compiler_params=pltpu.CompilerParams(
            dimension_semantics=("arbitrary", "arbitrary")),
    )(te, valid, isnew, rid, runex, meta, h, Wd, bd)


def kernel(x, gate_w, gate_b, Wg, bg, Wu, bu, Wd, bd):
    b, s, d = x.shape
    T = b * s
    xf = x.reshape(T, d)

    logits = xf @ gate_w.T + gate_b
    probs = jax.nn.softmax(logits.astype(jnp.float32), axis=1)
    rw, sel = lax.top_k(probs, TOP_K)
    rw = rw / jnp.sum(rw, axis=-1, keepdims=True)

    A = T * TOP_K
    onehot = jnp.sum(
        (sel[:, :, None] == jnp.arange(NUM_EXPERTS)[None, None, :])
        .astype(jnp.float32), axis=1)
    tri = (jnp.arange(T)[:, None] > jnp.arange(T)[None, :]).astype(jnp.float32)
    rank_before = (tri @ onehot).astype(jnp.int32)
    counts = jnp.sum(onehot, axis=0).astype(jnp.int32)
    pad_counts = ((counts + TM - 1) // TM) * TM
    pad_end = jnp.cumsum(pad_counts)
    pad_off = jnp.concatenate([jnp.zeros((1,), jnp.int32), pad_end[:-1]])
    pos = pad_off[sel] + jnp.take_along_axis(rank_before, sel, axis=1)
    pos = pos.reshape(-1)
    tok = (jnp.arange(A, dtype=jnp.int32) // TOP_K)
    perm_tok = jnp.zeros((P_CAP,), jnp.int32).at[pos].set(tok)
    xg = xf[perm_tok].astype(jnp.bfloat16)

    tile_start = jnp.arange(NT, dtype=jnp.int32) * TM
    last_row = pad_end[-1] - 1
    te = jnp.searchsorted(pad_end, jnp.minimum(tile_start, last_row),
                          side='right').astype(jnp.int32)
    valid = (tile_start < pad_end[-1]).astype(jnp.int32)
    isnew = jnp.concatenate([jnp.ones((1,), jnp.int32),
                             (te[1:] != te[:-1]).astype(jnp.int32)])
    rid = (jnp.cumsum(isnew) - 1).astype(jnp.int32)
    nrun = rid[-1] + 1
    runex = jnp.zeros((NUM_EXPERTS,), jnp.int32).at[rid].set(te)
    meta = jnp.stack([NF * nrun, nrun]).astype(jnp.int32)

    sched = (te, valid, isnew, rid, runex, meta)
    y = _grouped_mlp(sched, xg,
                     Wg, bg.reshape(NUM_EXPERTS * NF, 1, FB),
                     Wu, bu.reshape(NUM_EXPERTS * NF, 1, FB),
                     Wd, bd.reshape(NUM_EXPERTS, 1, D_MODEL))

    p = pos.reshape(T, TOP_K)
    final = rw[:, 0:1] * y[p[:, 0]] + rw[:, 1:2] * y[p[:, 1]]
    return final.reshape(b, s, d), logits
